# baseline (speedup 1.0000x reference)
import functools

import jax
import jax.numpy as jnp
from jax import lax
from jax.experimental import pallas as pl
from jax.experimental.pallas import tpu as pltpu, tpu_sc as plsc


_NC = 2
_NS = 16
_NW = _NC * _NS
_CHUNK = 8
_LANES = 16


def kernel(tensor_in, pos_table):
    batch, seq, dim = tensor_in.shape
    rows = batch * seq
    per_w = rows // _NW
    nchunks = per_w // _CHUNK
    x2d = tensor_in.reshape(rows, dim)
    mesh = plsc.VectorSubcoreMesh(core_axis_name="c", subcore_axis_name="s")

    @functools.partial(
        pl.kernel,
        out_type=jax.ShapeDtypeStruct((rows, dim), jnp.float32),
        mesh=mesh,
        scratch_types=[
            pltpu.VMEM((3, _CHUNK, dim), jnp.float32),
            pltpu.VMEM((3, _CHUNK, dim), jnp.float32),
            pltpu.SemaphoreType.DMA,
            pltpu.SemaphoreType.DMA,
            pltpu.SemaphoreType.DMA,
            pltpu.SemaphoreType.DMA,
            pltpu.SemaphoreType.DMA,
            pltpu.SemaphoreType.DMA,
            pltpu.SemaphoreType.DMA,
            pltpu.SemaphoreType.DMA,
            pltpu.SemaphoreType.DMA,
        ],
    )
    def sc_add(x_hbm, tab_hbm, out_hbm, tbuf, pbuf,
               ts0, ts1, ts2, ps0, ps1, ps2, os0, os1, os2):
        tsem = (ts0, ts1, ts2)
        psem = (ps0, ps1, ps2)
        osem = (os0, os1, os2)
        wid = lax.axis_index("s") * _NC + lax.axis_index("c")
        base = wid * per_w
        sbase = base % seq

        def start_in(k):
            s = k % 3
            r = base + k * _CHUNK
            sr = sbase + k * _CHUNK
            dt = pltpu.async_copy(x_hbm.at[pl.ds(r, _CHUNK)], tbuf.at[s], tsem[s])
            dp = pltpu.async_copy(tab_hbm.at[pl.ds(sr, _CHUNK)], pbuf.at[s], psem[s])
            return dt, dp

        ind = [None] * nchunks
        outd = [None] * nchunks
        ind[0] = start_in(0)
        ind[1] = start_in(1)
        for k in range(nchunks):
            s = k % 3
            if k + 2 < nchunks:
                if k - 1 >= 0:
                    outd[k - 1].wait()
                ind[k + 2] = start_in(k + 2)
            ind[k][0].wait()
            ind[k][1].wait()
            for row in range(_CHUNK):
                @plsc.parallel_loop(0, dim, _LANES, unroll=8)
                def _vec_body(j, _s=s, _row=row):
                    plsc.addupdate(
                        tbuf.at[_s, _row, pl.ds(j, _LANES)],
                        pbuf[_s, _row, pl.ds(j, _LANES)],
                    )
            r = base + k * _CHUNK
            outd[k] = pltpu.async_copy(tbuf.at[s], out_hbm.at[pl.ds(r, _CHUNK)], osem[s])
        outd[nchunks - 3].wait()
        outd[nchunks - 2].wait()
        outd[nchunks - 1].wait()

    out2d = sc_add(x2d, pos_table)
    return out2d.reshape(batch, seq, dim)


# speedup vs baseline: 1.0692x; 1.0692x over previous
import functools

import jax
import jax.numpy as jnp
from jax import lax
from jax.experimental import pallas as pl
from jax.experimental.pallas import tpu as pltpu, tpu_sc as plsc


_NC = 2
_NS = 16
_NW = _NC * _NS
_CHUNK = 8
_LANES = 16


def kernel(tensor_in, pos_table):
    batch, seq, dim = tensor_in.shape
    rows = batch * seq
    seq_per_w = seq // _NW
    ntab = seq_per_w // _CHUNK
    nten = ntab * batch
    x2d = tensor_in.reshape(rows, dim)
    mesh = plsc.VectorSubcoreMesh(core_axis_name="c", subcore_axis_name="s")

    @functools.partial(
        pl.kernel,
        out_type=jax.ShapeDtypeStruct((rows, dim), jnp.float32),
        mesh=mesh,
        scratch_types=[
            pltpu.VMEM((3, _CHUNK, dim), jnp.float32),
            pltpu.VMEM((2, _CHUNK, dim), jnp.float32),
            pltpu.SemaphoreType.DMA,
            pltpu.SemaphoreType.DMA,
            pltpu.SemaphoreType.DMA,
            pltpu.SemaphoreType.DMA,
            pltpu.SemaphoreType.DMA,
            pltpu.SemaphoreType.DMA,
            pltpu.SemaphoreType.DMA,
            pltpu.SemaphoreType.DMA,
        ],
    )
    def sc_add(x_hbm, tab_hbm, out_hbm, tbuf, pbuf,
               ts0, ts1, ts2, ps0, ps1, os0, os1, os2):
        tsem = (ts0, ts1, ts2)
        psem = (ps0, ps1)
        osem = (os0, os1, os2)
        wid = lax.axis_index("s") * _NC + lax.axis_index("c")
        sbase = wid * seq_per_w

        def row0(j):
            return sbase + (j % batch) * seq + (j // batch) * _CHUNK

        def start_t(j):
            s = j % 3
            return pltpu.async_copy(
                x_hbm.at[pl.ds(row0(j), _CHUNK)], tbuf.at[s], tsem[s])

        def start_p(tk):
            s = tk % 2
            return pltpu.async_copy(
                tab_hbm.at[pl.ds(sbase + tk * _CHUNK, _CHUNK)], pbuf.at[s], psem[s])

        pind = {0: start_p(0)}
        tind = {0: start_t(0), 1: start_t(1)}
        outd = {}
        for j in range(nten):
            s = j % 3
            tk = j // batch
            if j % batch == 0 and tk + 1 < ntab:
                pind[tk + 1] = start_p(tk + 1)
            if j + 2 < nten:
                if j - 1 >= 0:
                    outd[j - 1].wait()
                tind[j + 2] = start_t(j + 2)
            tind[j].wait()
            if j % batch == 0:
                pind[tk].wait()
            for row in range(_CHUNK):
                @plsc.parallel_loop(0, dim, _LANES, unroll=8)
                def _vec_body(col, _s=s, _ps=tk % 2, _row=row):
                    plsc.addupdate(
                        tbuf.at[_s, _row, pl.ds(col, _LANES)],
                        pbuf[_ps, _row, pl.ds(col, _LANES)],
                    )
            outd[j] = pltpu.async_copy(
                tbuf.at[s], out_hbm.at[pl.ds(row0(j), _CHUNK)], osem[s])
        for j in range(max(0, nten - 3), nten):
            outd[j].wait()

    out2d = sc_add(x2d, pos_table)
    return out2d.reshape(batch, seq, dim)


# speedup vs baseline: 2.4498x; 2.2912x over previous
import jax
import jax.numpy as jnp
from jax.experimental import pallas as pl


_SEQ_BLOCK = 1024


def _add_block(tensor_ref, pos_ref, out_ref):
    out_ref[...] = tensor_ref[...] + pos_ref[...]


def kernel(tensor_in, pos_table):
    batch, seq, dim = tensor_in.shape
    grid = (seq // _SEQ_BLOCK, batch)
    return pl.pallas_call(
        _add_block,
        grid=grid,
        in_specs=[
            pl.BlockSpec((1, _SEQ_BLOCK, dim), lambda i, j: (j, i, 0)),
            pl.BlockSpec((_SEQ_BLOCK, dim), lambda i, j: (i, 0)),
        ],
        out_specs=pl.BlockSpec((1, _SEQ_BLOCK, dim), lambda i, j: (j, i, 0)),
        out_shape=jax.ShapeDtypeStruct(tensor_in.shape, tensor_in.dtype),
    )(tensor_in, pos_table)


# speedup vs baseline: 2.4604x; 1.0043x over previous
import jax
from jax.experimental import pallas as pl


_SEQ_BLOCK = 1024


def _add_block(tensor_ref, pos_ref, out_ref):
    out_ref[...] = tensor_ref[...] + pos_ref[...]


def kernel(tensor_in, pos_table):
    batch, seq, dim = tensor_in.shape
    grid = (seq // _SEQ_BLOCK, batch)
    return pl.pallas_call(
        _add_block,
        grid=grid,
        in_specs=[
            pl.BlockSpec((1, _SEQ_BLOCK, dim), lambda i, j: (j, i, 0)),
            pl.BlockSpec((_SEQ_BLOCK, dim), lambda i, j: (i, 0)),
        ],
        out_specs=pl.BlockSpec((1, _SEQ_BLOCK, dim), lambda i, j: (j, i, 0)),
        out_shape=jax.ShapeDtypeStruct(tensor_in.shape, tensor_in.dtype),
    )(tensor_in, pos_table)


# speedup vs baseline: 2.4869x; 1.0108x over previous
import jax
from jax.experimental import pallas as pl


_SEQ_BLOCK = 1024


def _add_block(tensor_ref, pos_ref, out_ref):
    i = pl.program_id(0)
    out_ref[...] = tensor_ref[...] + pos_ref[pl.ds(i * _SEQ_BLOCK, _SEQ_BLOCK), :]


def kernel(tensor_in, pos_table):
    batch, seq, dim = tensor_in.shape
    grid = (seq // _SEQ_BLOCK, batch)
    return pl.pallas_call(
        _add_block,
        grid=grid,
        in_specs=[
            pl.BlockSpec((1, _SEQ_BLOCK, dim), lambda i, j: (j, i, 0)),
            pl.BlockSpec((seq, dim), lambda i, j: (0, 0)),
        ],
        out_specs=pl.BlockSpec((1, _SEQ_BLOCK, dim), lambda i, j: (j, i, 0)),
        out_shape=jax.ShapeDtypeStruct(tensor_in.shape, tensor_in.dtype),
    )(tensor_in, pos_table)
